# baseline (speedup 1.0000x reference)
import jax
import jax.numpy as jnp
from jax import lax
from jax.experimental import pallas as pl
from jax.experimental.pallas import tpu as pltpu
from jax.experimental.pallas import tpu_sc as plsc

_N = 10000
_E = 320000
_NC = 2
_NS = 16
_NW = _NC * _NS
_CHUNK = 64
_CPW = 162
_NBUF = 3
_EPAD = _NW * _CPW * _CHUNK
_RPT = 632
_NPAD = _NS * _RPT


def _edge_pass(xa, eb, src, dst, d):
    ngrp = d // 16
    mesh = plsc.VectorSubcoreMesh(core_axis_name="c", subcore_axis_name="s")

    def body(xa_hbm, eb_hbm, src_hbm, dst_hbm, out_hbm,
             src_v0, dst_v0, rows_v0, eb_v0, gsem0, esem0, ssem0,
             src_v1, dst_v1, rows_v1, eb_v1, gsem1, esem1, ssem1,
             src_v2, dst_v2, rows_v2, eb_v2, gsem2, esem2, ssem2,
             agg_sh):
        c = lax.axis_index("c")
        s = lax.axis_index("s")
        wid = s * _NC + c
        bufs = ((src_v0, dst_v0, rows_v0, eb_v0, gsem0, esem0, ssem0),
                (src_v1, dst_v1, rows_v1, eb_v1, gsem1, esem1, ssem1),
                (src_v2, dst_v2, rows_v2, eb_v2, gsem2, esem2, ssem2))

        z = jnp.zeros((16,), jnp.float32)

        def zbody(j, carry):
            for k in range(ngrp):
                rows_v2[j, pl.ds(k * 16, 16)] = z
            return carry

        lax.fori_loop(0, _CHUNK, zbody, 0)
        row0 = s * _RPT
        zfull, zrem = _RPT // _CHUNK, _RPT % _CHUNK
        for r in range(zfull):
            pltpu.sync_copy(rows_v2, agg_sh.at[pl.ds(row0 + r * _CHUNK, _CHUNK)])
        if zrem:
            pltpu.sync_copy(rows_v2.at[pl.ds(0, zrem)],
                            agg_sh.at[pl.ds(row0 + zfull * _CHUNK, zrem)])
        base0 = wid * _CHUNK
        pltpu.sync_copy(dst_hbm.at[pl.ds(base0, _CHUNK)], dst_v2)
        pltpu.async_copy(rows_v2, agg_sh.at[dst_v2], ssem2, add=True)
        plsc.subcore_barrier()

        def prefetch(i, b):
            sv, dv, rv, ev, gs, es, ss = bufs[b]
            pltpu.make_async_copy(rv, agg_sh.at[dv], ss).wait()
            base = (wid + i * _NW) * _CHUNK
            pltpu.sync_copy(src_hbm.at[pl.ds(base, _CHUNK)], sv)
            pltpu.sync_copy(dst_hbm.at[pl.ds(base, _CHUNK)], dv)
            pltpu.async_copy(xa_hbm.at[sv], rv, gs)
            pltpu.async_copy(eb_hbm.at[pl.ds(base, _CHUNK)], ev, es)

        def consume(b):
            sv, dv, rv, ev, gs, es, ss = bufs[b]
            pltpu.make_async_copy(xa_hbm.at[sv], rv, gs).wait()
            pltpu.make_async_copy(eb_hbm.at[pl.ds(0, _CHUNK)], ev, es).wait()

            def mbody(j, mcarry):
                for k in range(ngrp):
                    sl = pl.ds(k * 16, 16)
                    rv[j, sl] = rv[j, sl] * ev[j, sl]
                return mcarry

            lax.fori_loop(0, _CHUNK, mbody, 0)
            pltpu.async_copy(rv, agg_sh.at[dv], ss, add=True)

        sv, dv, rv, ev, gs, es, ss = bufs[0]
        base = wid * _CHUNK
        pltpu.sync_copy(src_hbm.at[pl.ds(base, _CHUNK)], sv)
        pltpu.sync_copy(dst_hbm.at[pl.ds(base, _CHUNK)], dv)
        pltpu.async_copy(xa_hbm.at[sv], rv, gs)
        pltpu.async_copy(eb_hbm.at[pl.ds(base, _CHUNK)], ev, es)
        sv, dv, rv, ev, gs, es, ss = bufs[1]
        base = (wid + _NW) * _CHUNK
        pltpu.sync_copy(src_hbm.at[pl.ds(base, _CHUNK)], sv)
        pltpu.sync_copy(dst_hbm.at[pl.ds(base, _CHUNK)], dv)
        pltpu.async_copy(xa_hbm.at[sv], rv, gs)
        pltpu.async_copy(eb_hbm.at[pl.ds(base, _CHUNK)], ev, es)

        def tri_body(tt, carry):
            for b in range(_NBUF):
                g = _NBUF * tt + b
                consume(b)
                prefetch(jnp.minimum(g + 2, _CPW - 1), (b + 2) % _NBUF)
            return carry

        lax.fori_loop(0, _CPW // _NBUF, tri_body, 0)

        for b in (0, 1):
            sv, dv, rv, ev, gs, es, ss = bufs[b]
            pltpu.make_async_copy(xa_hbm.at[sv], rv, gs).wait()
            pltpu.make_async_copy(eb_hbm.at[pl.ds(0, _CHUNK)], ev, es).wait()
        sv, dv, rv, ev, gs, es, ss = bufs[2]
        pltpu.make_async_copy(rv, agg_sh.at[dv], ss).wait()

        plsc.subcore_barrier()
        pltpu.sync_copy(agg_sh.at[pl.ds(row0, _RPT)],
                        out_hbm.at[c, pl.ds(row0, _RPT)])

    buf_types = [
        pltpu.VMEM((_CHUNK,), jnp.int32),
        pltpu.VMEM((_CHUNK,), jnp.int32),
        pltpu.VMEM((_CHUNK, d), jnp.float32),
        pltpu.VMEM((_CHUNK, d), jnp.float32),
        pltpu.SemaphoreType.DMA,
        pltpu.SemaphoreType.DMA,
        pltpu.SemaphoreType.DMA,
    ]
    call = pl.kernel(
        body,
        out_type=jax.ShapeDtypeStruct((_NC, _NPAD, d), jnp.float32),
        mesh=mesh,
        compiler_params=pltpu.CompilerParams(use_tc_tiling_on_sc=(d == 128)),
        scratch_types=buf_types * _NBUF + [
            pltpu.VMEM_SHARED((_NPAD, d), jnp.float32),
        ],
    )
    return call(xa, eb, src, dst)


_EBLK = 2048


def _eb_one(attr_p, wb):
    dd = int(wb.shape[1])

    def body(attr_ref, w_ref, o_ref):
        o_ref[...] = jnp.dot(attr_ref[...], w_ref[...],
                             preferred_element_type=jnp.float32)

    return pl.pallas_call(
        body,
        grid=(_EPAD // _EBLK,),
        in_specs=[
            pl.BlockSpec((_EBLK, 16), lambda i: (i, 0)),
            pl.BlockSpec((16, dd), lambda i: (0, 0)),
        ],
        out_specs=pl.BlockSpec((_EBLK, dd), lambda i: (i, 0)),
        out_shape=jax.ShapeDtypeStruct((_EPAD, dd), jnp.float32),
    )(attr_p, wb)


_TB = 2000


def _mm_pair(xin, wa, ws):
    din = int(xin.shape[1])
    da, dsf = int(wa.shape[1]), int(ws.shape[1])

    def body(x_ref, wa_ref, ws_ref, oa_ref, os_ref):
        x0 = x_ref[:, 0:1]
        oa_ref[...] = x0 * wa_ref[0:1, :]
        os_ref[...] = x0 * ws_ref[0:1, :]

    return pl.pallas_call(
        body,
        grid=(_N // _TB,),
        in_specs=[
            pl.BlockSpec((_TB, din), lambda i: (i, 0)),
            pl.BlockSpec((din, da), lambda i: (0, 0)),
            pl.BlockSpec((din, dsf), lambda i: (0, 0)),
        ],
        out_specs=[
            pl.BlockSpec((_TB, da), lambda i: (i, 0)),
            pl.BlockSpec((_TB, dsf), lambda i: (i, 0)),
        ],
        out_shape=[
            jax.ShapeDtypeStruct((_N, da), jnp.float32),
            jax.ShapeDtypeStruct((_N, dsf), jnp.float32),
        ],
    )(xin, wa, ws)


def _gate_block(h):
    sg = jax.nn.gelu(h[:, 0:32])
    st = jnp.tanh(h[:, 32:64])
    g = jax.nn.sigmoid(h[:, 64:80])
    row = lax.broadcasted_iota(jnp.int32, (16, 48), 0)
    col = lax.broadcasted_iota(jnp.int32, (16, 48), 1)
    r = (row == col // 3).astype(jnp.float32)
    gm = jnp.dot(g, r, preferred_element_type=jnp.float32, precision=lax.Precision.HIGHEST)
    return jnp.concatenate([sg, st, h[:, 80:128] * gm], axis=1)


def _layer_tc(xs_prev, agg, wa, ws):
    da, dsf = int(wa.shape[1]), int(ws.shape[1])

    def body(xs_ref, agg_ref, wa_ref, ws_ref, oa_ref, os_ref):
        h = (xs_ref[...] + agg_ref[0] + agg_ref[1]) / jnp.sqrt(jnp.float32(1.5))
        xv = _gate_block(h)
        oa_ref[...] = jnp.dot(xv, wa_ref[...], preferred_element_type=jnp.float32)
        os_ref[...] = jnp.dot(xv, ws_ref[...], preferred_element_type=jnp.float32)

    return pl.pallas_call(
        body,
        grid=(_N // _TB,),
        in_specs=[
            pl.BlockSpec((_TB, 128), lambda i: (i, 0)),
            pl.BlockSpec((_NC, _TB, 128), lambda i: (0, i, 0)),
            pl.BlockSpec((112, da), lambda i: (0, 0)),
            pl.BlockSpec((112, dsf), lambda i: (0, 0)),
        ],
        out_specs=[
            pl.BlockSpec((_TB, da), lambda i: (i, 0)),
            pl.BlockSpec((_TB, dsf), lambda i: (i, 0)),
        ],
        out_shape=[
            jax.ShapeDtypeStruct((_N, da), jnp.float32),
            jax.ShapeDtypeStruct((_N, dsf), jnp.float32),
        ],
    )(xs_prev, agg, wa, ws)


def _final_combine(xs4, agg4):
    def body(xs_ref, agg_ref, o_ref):
        o_ref[...] = (xs_ref[...] + agg_ref[0] + agg_ref[1]) / jnp.sqrt(
            jnp.float32(1.5))

    return pl.pallas_call(
        body,
        grid=(_N // _TB,),
        in_specs=[
            pl.BlockSpec((_TB, 16), lambda i: (i, 0)),
            pl.BlockSpec((_NC, _TB, 16), lambda i: (0, i, 0)),
        ],
        out_specs=pl.BlockSpec((_TB, 16), lambda i: (i, 0)),
        out_shape=jax.ShapeDtypeStruct((_N, 16), jnp.float32),
    )(xs4, agg4)


def kernel(x, edge_src, edge_dst, edge_attr,
           W_self_0, W_a_0, W_b_0,
           W_self_1, W_a_1, W_b_1,
           W_self_2, W_a_2, W_b_2,
           W_self_3, W_a_3, W_b_3,
           W_self_4, W_a_4, W_b_4):
    pad_e = _EPAD - _E
    pad_idx = jnp.arange(pad_e, dtype=jnp.int32) % _N
    src_p = jnp.concatenate([edge_src, pad_idx])
    dst_p = jnp.concatenate([edge_dst, pad_idx])
    attr_p = jnp.pad(edge_attr, ((0, pad_e), (0, 7)))
    wb = [jnp.pad(w, ((0, 7), (0, 0))) for w in (W_b_0, W_b_1, W_b_2, W_b_3)]
    wb4 = jnp.pad(W_b_4, ((0, 7), (0, 9)))
    wa4 = jnp.pad(W_a_4, ((0, 0), (0, 9)))
    ws4 = jnp.pad(W_self_4, ((0, 0), (0, 9)))
    x_p = jnp.pad(x, ((0, 0), (0, 7)))
    wa0 = jnp.pad(W_a_0, ((0, 7), (0, 0)))
    ws0 = jnp.pad(W_self_0, ((0, 7), (0, 0)))

    ebs = [_eb_one(attr_p, w) for w in (wb[0], wb[1], wb[2], wb[3], wb4)]

    xa, xs = _mm_pair(x_p, wa0, ws0)
    layer_ws = [(W_a_1, W_self_1), (W_a_2, W_self_2), (W_a_3, W_self_3),
                (wa4, ws4)]
    for i in range(4):
        agg = _edge_pass(xa, ebs[i], src_p, dst_p, 128)
        xa, xs = _layer_tc(xs, agg, layer_ws[i][0], layer_ws[i][1])
    agg4 = _edge_pass(xa, ebs[4], src_p, dst_p, 16)
    out16 = _final_combine(xs, agg4)
    return out16[:, :7]


# speedup vs baseline: 1.0216x; 1.0216x over previous
import jax
import jax.numpy as jnp
from jax import lax
from jax.experimental import pallas as pl
from jax.experimental.pallas import tpu as pltpu
from jax.experimental.pallas import tpu_sc as plsc

_N = 10000
_E = 320000
_NC = 2
_NS = 16
_NW = _NC * _NS
_CHUNK = 96
_CPW = 106
_EPAD = _NW * _CPW * _CHUNK
_RPT = 632
_NPAD = _NS * _RPT


def _edge_pass(xa, eb, src, dst, d):
    ngrp = d // 16
    mesh = plsc.VectorSubcoreMesh(core_axis_name="c", subcore_axis_name="s")

    def body(xa_hbm, eb_hbm, src_hbm, dst_hbm, out_hbm,
             src_v0, dst_v0, rows_v0, eb_v0, gsem0, esem0, ssem0,
             src_v1, dst_v1, rows_v1, eb_v1, gsem1, esem1, ssem1,
             agg_sh):
        c = lax.axis_index("c")
        s = lax.axis_index("s")
        wid = s * _NC + c
        bufs = ((src_v0, dst_v0, rows_v0, eb_v0, gsem0, esem0, ssem0),
                (src_v1, dst_v1, rows_v1, eb_v1, gsem1, esem1, ssem1))

        z = jnp.zeros((16,), jnp.float32)

        def zbody(j, carry):
            for k in range(ngrp):
                rows_v0[j, pl.ds(k * 16, 16)] = z
            return carry

        lax.fori_loop(0, _CHUNK, zbody, 0)
        row0 = s * _RPT
        zfull, zrem = _RPT // _CHUNK, _RPT % _CHUNK
        for r in range(zfull):
            pltpu.sync_copy(rows_v0, agg_sh.at[pl.ds(row0 + r * _CHUNK, _CHUNK)])
        if zrem:
            pltpu.sync_copy(rows_v0.at[pl.ds(0, zrem)],
                            agg_sh.at[pl.ds(row0 + zfull * _CHUNK, zrem)])

        def zbody1(j, carry):
            for k in range(ngrp):
                rows_v1[j, pl.ds(k * 16, 16)] = z
            return carry

        lax.fori_loop(0, _CHUNK, zbody1, 0)
        pltpu.sync_copy(dst_hbm.at[pl.ds(wid * _CHUNK, _CHUNK)], dst_v1)
        pltpu.async_copy(rows_v1, agg_sh.at[dst_v1], ssem1, add=True)
        plsc.subcore_barrier()

        def prefetch(i, b):
            sv, dv, rv, ev, gs, es, ss = bufs[b]
            base = (wid + i * _NW) * _CHUNK
            pltpu.sync_copy(src_hbm.at[pl.ds(base, _CHUNK)], sv)
            pltpu.sync_copy(dst_hbm.at[pl.ds(base, _CHUNK)], dv)
            pltpu.async_copy(xa_hbm.at[sv], rv, gs)
            pltpu.async_copy(eb_hbm.at[pl.ds(base, _CHUNK)], ev, es)

        def consume(b):
            sv, dv, rv, ev, gs, es, ss = bufs[b]
            pltpu.make_async_copy(xa_hbm.at[sv], rv, gs).wait()
            pltpu.make_async_copy(eb_hbm.at[pl.ds(0, _CHUNK)], ev, es).wait()

            def mbody(j, mcarry):
                for u in range(2):
                    for k in range(ngrp):
                        sl = pl.ds(k * 16, 16)
                        rv[2 * j + u, sl] = rv[2 * j + u, sl] * ev[2 * j + u, sl]
                return mcarry

            lax.fori_loop(0, _CHUNK // 2, mbody, 0)
            pltpu.async_copy(rv, agg_sh.at[dv], ss, add=True)

        def scwait(b):
            sv, dv, rv, ev, gs, es, ss = bufs[b]
            pltpu.make_async_copy(rv, agg_sh.at[dv], ss).wait()

        prefetch(jnp.int32(0), 0)

        def pair_body(p, carry):
            g = 2 * p
            scwait(1)
            prefetch(g + 1, 1)
            consume(0)
            consume(1)
            scwait(0)
            prefetch(jnp.minimum(g + 2, _CPW - 1), 0)
            return carry

        lax.fori_loop(0, _CPW // 2, pair_body, 0)
        sv, dv, rv, ev, gs, es, ss = bufs[0]
        pltpu.make_async_copy(xa_hbm.at[sv], rv, gs).wait()
        pltpu.make_async_copy(eb_hbm.at[pl.ds(0, _CHUNK)], ev, es).wait()
        scwait(1)

        plsc.subcore_barrier()
        pltpu.sync_copy(agg_sh.at[pl.ds(row0, _RPT)],
                        out_hbm.at[c, pl.ds(row0, _RPT)])

    buf_types = [
        pltpu.VMEM((_CHUNK,), jnp.int32),
        pltpu.VMEM((_CHUNK,), jnp.int32),
        pltpu.VMEM((_CHUNK, d), jnp.float32),
        pltpu.VMEM((_CHUNK, d), jnp.float32),
        pltpu.SemaphoreType.DMA,
        pltpu.SemaphoreType.DMA,
        pltpu.SemaphoreType.DMA,
    ]
    call = pl.kernel(
        body,
        out_type=jax.ShapeDtypeStruct((_NC, _NPAD, d), jnp.float32),
        mesh=mesh,
        compiler_params=pltpu.CompilerParams(use_tc_tiling_on_sc=(d == 128)),
        scratch_types=buf_types + buf_types + [
            pltpu.VMEM_SHARED((_NPAD, d), jnp.float32),
        ],
    )
    return call(xa, eb, src, dst)


_EBLK = 2048


def _eb_one(attr_p, wb):
    dd = int(wb.shape[1])

    def body(attr_ref, w_ref, o_ref):
        o_ref[...] = jnp.dot(attr_ref[...], w_ref[...],
                             preferred_element_type=jnp.float32)

    return pl.pallas_call(
        body,
        grid=(_EPAD // _EBLK,),
        in_specs=[
            pl.BlockSpec((_EBLK, 16), lambda i: (i, 0)),
            pl.BlockSpec((16, dd), lambda i: (0, 0)),
        ],
        out_specs=pl.BlockSpec((_EBLK, dd), lambda i: (i, 0)),
        out_shape=jax.ShapeDtypeStruct((_EPAD, dd), jnp.float32),
    )(attr_p, wb)


_TB = 2000


def _mm_pair(xin, wa, ws):
    din = int(xin.shape[1])
    da, dsf = int(wa.shape[1]), int(ws.shape[1])

    def body(x_ref, wa_ref, ws_ref, oa_ref, os_ref):
        x0 = x_ref[:, 0:1]
        oa_ref[...] = x0 * wa_ref[0:1, :]
        os_ref[...] = x0 * ws_ref[0:1, :]

    return pl.pallas_call(
        body,
        grid=(_N // _TB,),
        in_specs=[
            pl.BlockSpec((_TB, din), lambda i: (i, 0)),
            pl.BlockSpec((din, da), lambda i: (0, 0)),
            pl.BlockSpec((din, dsf), lambda i: (0, 0)),
        ],
        out_specs=[
            pl.BlockSpec((_TB, da), lambda i: (i, 0)),
            pl.BlockSpec((_TB, dsf), lambda i: (i, 0)),
        ],
        out_shape=[
            jax.ShapeDtypeStruct((_N, da), jnp.float32),
            jax.ShapeDtypeStruct((_N, dsf), jnp.float32),
        ],
    )(xin, wa, ws)


def _gate_block(h):
    sg = jax.nn.gelu(h[:, 0:32])
    st = jnp.tanh(h[:, 32:64])
    g = jax.nn.sigmoid(h[:, 64:80])
    row = lax.broadcasted_iota(jnp.int32, (16, 48), 0)
    col = lax.broadcasted_iota(jnp.int32, (16, 48), 1)
    r = (row == col // 3).astype(jnp.float32)
    gm = jnp.dot(g, r, preferred_element_type=jnp.float32, precision=lax.Precision.HIGHEST)
    return jnp.concatenate([sg, st, h[:, 80:128] * gm], axis=1)


def _layer_tc(xs_prev, agg, wa, ws):
    da, dsf = int(wa.shape[1]), int(ws.shape[1])

    def body(xs_ref, agg_ref, wa_ref, ws_ref, oa_ref, os_ref):
        h = (xs_ref[...] + agg_ref[0] + agg_ref[1]) / jnp.sqrt(jnp.float32(1.5))
        xv = _gate_block(h)
        oa_ref[...] = jnp.dot(xv, wa_ref[...], preferred_element_type=jnp.float32)
        os_ref[...] = jnp.dot(xv, ws_ref[...], preferred_element_type=jnp.float32)

    return pl.pallas_call(
        body,
        grid=(_N // _TB,),
        in_specs=[
            pl.BlockSpec((_TB, 128), lambda i: (i, 0)),
            pl.BlockSpec((_NC, _TB, 128), lambda i: (0, i, 0)),
            pl.BlockSpec((112, da), lambda i: (0, 0)),
            pl.BlockSpec((112, dsf), lambda i: (0, 0)),
        ],
        out_specs=[
            pl.BlockSpec((_TB, da), lambda i: (i, 0)),
            pl.BlockSpec((_TB, dsf), lambda i: (i, 0)),
        ],
        out_shape=[
            jax.ShapeDtypeStruct((_N, da), jnp.float32),
            jax.ShapeDtypeStruct((_N, dsf), jnp.float32),
        ],
    )(xs_prev, agg, wa, ws)


def _final_combine(xs4, agg4):
    def body(xs_ref, agg_ref, o_ref):
        o_ref[...] = (xs_ref[...] + agg_ref[0] + agg_ref[1]) / jnp.sqrt(
            jnp.float32(1.5))

    return pl.pallas_call(
        body,
        grid=(_N // _TB,),
        in_specs=[
            pl.BlockSpec((_TB, 16), lambda i: (i, 0)),
            pl.BlockSpec((_NC, _TB, 16), lambda i: (0, i, 0)),
        ],
        out_specs=pl.BlockSpec((_TB, 16), lambda i: (i, 0)),
        out_shape=jax.ShapeDtypeStruct((_N, 16), jnp.float32),
    )(xs4, agg4)


def kernel(x, edge_src, edge_dst, edge_attr,
           W_self_0, W_a_0, W_b_0,
           W_self_1, W_a_1, W_b_1,
           W_self_2, W_a_2, W_b_2,
           W_self_3, W_a_3, W_b_3,
           W_self_4, W_a_4, W_b_4):
    pad_e = _EPAD - _E
    pad_idx = jnp.arange(pad_e, dtype=jnp.int32) % _N
    src_p = jnp.concatenate([edge_src, pad_idx])
    dst_p = jnp.concatenate([edge_dst, pad_idx])
    attr_p = jnp.pad(edge_attr, ((0, pad_e), (0, 7)))
    wb = [jnp.pad(w, ((0, 7), (0, 0))) for w in (W_b_0, W_b_1, W_b_2, W_b_3)]
    wb4 = jnp.pad(W_b_4, ((0, 7), (0, 9)))
    wa4 = jnp.pad(W_a_4, ((0, 0), (0, 9)))
    ws4 = jnp.pad(W_self_4, ((0, 0), (0, 9)))
    x_p = jnp.pad(x, ((0, 0), (0, 7)))
    wa0 = jnp.pad(W_a_0, ((0, 7), (0, 0)))
    ws0 = jnp.pad(W_self_0, ((0, 7), (0, 0)))

    ebs = [_eb_one(attr_p, w) for w in (wb[0], wb[1], wb[2], wb[3], wb4)]

    xa, xs = _mm_pair(x_p, wa0, ws0)
    layer_ws = [(W_a_1, W_self_1), (W_a_2, W_self_2), (W_a_3, W_self_3),
                (wa4, ws4)]
    for i in range(4):
        agg = _edge_pass(xa, ebs[i], src_p, dst_p, 128)
        xa, xs = _layer_tc(xs, agg, layer_ws[i][0], layer_ws[i][1])
    agg4 = _edge_pass(xa, ebs[4], src_p, dst_p, 16)
    out16 = _final_combine(xs, agg4)
    return out16[:, :7]


# speedup vs baseline: 1.0545x; 1.0322x over previous
import jax
import jax.numpy as jnp
from jax import lax
from jax.experimental import pallas as pl
from jax.experimental.pallas import tpu as pltpu
from jax.experimental.pallas import tpu_sc as plsc

_N = 10000
_E = 320000
_NC = 2
_NS = 16
_NW = _NC * _NS
_CHUNK = 96
_CPW = 106
_EPAD = _NW * _CPW * _CHUNK
_RPT = 632
_NPAD = _NS * _RPT


def _edge_pass(xa, eb, src, dst, d):
    ngrp = d // 16
    mesh = plsc.VectorSubcoreMesh(core_axis_name="c", subcore_axis_name="s")

    def body(xa_hbm, eb_hbm, src_hbm, dst_hbm, out_hbm,
             src_v0, dst_v0, rows_v0, eb_v0, gsem0, esem0,
             src_v1, dst_v1, rows_v1, eb_v1, gsem1, esem1,
             agg_sh):
        c = lax.axis_index("c")
        s = lax.axis_index("s")
        wid = s * _NC + c
        bufs = ((src_v0, dst_v0, rows_v0, eb_v0, gsem0, esem0),
                (src_v1, dst_v1, rows_v1, eb_v1, gsem1, esem1))

        z = jnp.zeros((16,), jnp.float32)

        def zbody(j, carry):
            for k in range(ngrp):
                rows_v0[j, pl.ds(k * 16, 16)] = z
            return carry

        lax.fori_loop(0, _CHUNK, zbody, 0)
        row0 = s * _RPT
        zfull, zrem = _RPT // _CHUNK, _RPT % _CHUNK
        for r in range(zfull):
            pltpu.sync_copy(rows_v0, agg_sh.at[pl.ds(row0 + r * _CHUNK, _CHUNK)])
        if zrem:
            pltpu.sync_copy(rows_v0.at[pl.ds(0, zrem)],
                            agg_sh.at[pl.ds(row0 + zfull * _CHUNK, zrem)])
        plsc.subcore_barrier()

        def prefetch(i, b):
            sv, dv, rv, ev, gs, es = bufs[b]
            base = (wid + i * _NW) * _CHUNK
            pltpu.sync_copy(src_hbm.at[pl.ds(base, _CHUNK)], sv)
            pltpu.sync_copy(dst_hbm.at[pl.ds(base, _CHUNK)], dv)
            pltpu.async_copy(xa_hbm.at[sv], rv, gs)
            pltpu.async_copy(eb_hbm.at[pl.ds(base, _CHUNK)], ev, es)

        def consume(b):
            sv, dv, rv, ev, gs, es = bufs[b]
            pltpu.make_async_copy(xa_hbm.at[sv], rv, gs).wait()
            pltpu.make_async_copy(eb_hbm.at[pl.ds(0, _CHUNK)], ev, es).wait()

            def mbody(j, mcarry):
                for u in range(2):
                    for k in range(ngrp):
                        sl = pl.ds(k * 16, 16)
                        rv[2 * j + u, sl] = rv[2 * j + u, sl] * ev[2 * j + u, sl]
                return mcarry

            lax.fori_loop(0, _CHUNK // 2, mbody, 0)
            pltpu.sync_copy(rv, agg_sh.at[dv], add=True)

        prefetch(jnp.int32(0), 0)

        def pair_body(p, carry):
            g = 2 * p
            prefetch(g + 1, 1)
            consume(0)
            prefetch(jnp.minimum(g + 2, _CPW - 1), 0)
            consume(1)
            return carry

        lax.fori_loop(0, _CPW // 2, pair_body, 0)
        sv, dv, rv, ev, gs, es = bufs[0]
        pltpu.make_async_copy(xa_hbm.at[sv], rv, gs).wait()
        pltpu.make_async_copy(eb_hbm.at[pl.ds(0, _CHUNK)], ev, es).wait()

        plsc.subcore_barrier()
        pltpu.sync_copy(agg_sh.at[pl.ds(row0, _RPT)],
                        out_hbm.at[c, pl.ds(row0, _RPT)])

    buf_types = [
        pltpu.VMEM((_CHUNK,), jnp.int32),
        pltpu.VMEM((_CHUNK,), jnp.int32),
        pltpu.VMEM((_CHUNK, d), jnp.float32),
        pltpu.VMEM((_CHUNK, d), jnp.float32),
        pltpu.SemaphoreType.DMA,
        pltpu.SemaphoreType.DMA,
    ]
    call = pl.kernel(
        body,
        out_type=jax.ShapeDtypeStruct((_NC, _NPAD, d), jnp.float32),
        mesh=mesh,
        compiler_params=pltpu.CompilerParams(use_tc_tiling_on_sc=(d == 128)),
        scratch_types=buf_types + buf_types + [
            pltpu.VMEM_SHARED((_NPAD, d), jnp.float32),
        ],
    )
    return call(xa, eb, src, dst)


_EBLK = 2048


def _eb_one(attr_p, wb):
    dd = int(wb.shape[1])

    def body(attr_ref, w_ref, o_ref):
        o_ref[...] = jnp.dot(attr_ref[...], w_ref[...],
                             preferred_element_type=jnp.float32)

    return pl.pallas_call(
        body,
        grid=(_EPAD // _EBLK,),
        in_specs=[
            pl.BlockSpec((_EBLK, 16), lambda i: (i, 0)),
            pl.BlockSpec((16, dd), lambda i: (0, 0)),
        ],
        out_specs=pl.BlockSpec((_EBLK, dd), lambda i: (i, 0)),
        out_shape=jax.ShapeDtypeStruct((_EPAD, dd), jnp.float32),
    )(attr_p, wb)


_TB = 2000


def _mm_pair(xin, wa, ws):
    din = int(xin.shape[1])
    da, dsf = int(wa.shape[1]), int(ws.shape[1])

    def body(x_ref, wa_ref, ws_ref, oa_ref, os_ref):
        x0 = x_ref[:, 0:1]
        oa_ref[...] = x0 * wa_ref[0:1, :]
        os_ref[...] = x0 * ws_ref[0:1, :]

    return pl.pallas_call(
        body,
        grid=(_N // _TB,),
        in_specs=[
            pl.BlockSpec((_TB, din), lambda i: (i, 0)),
            pl.BlockSpec((din, da), lambda i: (0, 0)),
            pl.BlockSpec((din, dsf), lambda i: (0, 0)),
        ],
        out_specs=[
            pl.BlockSpec((_TB, da), lambda i: (i, 0)),
            pl.BlockSpec((_TB, dsf), lambda i: (i, 0)),
        ],
        out_shape=[
            jax.ShapeDtypeStruct((_N, da), jnp.float32),
            jax.ShapeDtypeStruct((_N, dsf), jnp.float32),
        ],
    )(xin, wa, ws)


def _gate_block(h):
    sg = jax.nn.gelu(h[:, 0:32])
    st = jnp.tanh(h[:, 32:64])
    g = jax.nn.sigmoid(h[:, 64:80])
    row = lax.broadcasted_iota(jnp.int32, (16, 48), 0)
    col = lax.broadcasted_iota(jnp.int32, (16, 48), 1)
    r = (row == col // 3).astype(jnp.float32)
    gm = jnp.dot(g, r, preferred_element_type=jnp.float32, precision=lax.Precision.HIGHEST)
    return jnp.concatenate([sg, st, h[:, 80:128] * gm], axis=1)


def _layer_tc(xs_prev, agg, wa, ws):
    da, dsf = int(wa.shape[1]), int(ws.shape[1])

    def body(xs_ref, agg_ref, wa_ref, ws_ref, oa_ref, os_ref):
        h = (xs_ref[...] + agg_ref[0] + agg_ref[1]) / jnp.sqrt(jnp.float32(1.5))
        xv = _gate_block(h)
        oa_ref[...] = jnp.dot(xv, wa_ref[...], preferred_element_type=jnp.float32)
        os_ref[...] = jnp.dot(xv, ws_ref[...], preferred_element_type=jnp.float32)

    return pl.pallas_call(
        body,
        grid=(_N // _TB,),
        in_specs=[
            pl.BlockSpec((_TB, 128), lambda i: (i, 0)),
            pl.BlockSpec((_NC, _TB, 128), lambda i: (0, i, 0)),
            pl.BlockSpec((112, da), lambda i: (0, 0)),
            pl.BlockSpec((112, dsf), lambda i: (0, 0)),
        ],
        out_specs=[
            pl.BlockSpec((_TB, da), lambda i: (i, 0)),
            pl.BlockSpec((_TB, dsf), lambda i: (i, 0)),
        ],
        out_shape=[
            jax.ShapeDtypeStruct((_N, da), jnp.float32),
            jax.ShapeDtypeStruct((_N, dsf), jnp.float32),
        ],
    )(xs_prev, agg, wa, ws)


def _final_combine(xs4, agg4):
    def body(xs_ref, agg_ref, o_ref):
        o_ref[...] = (xs_ref[...] + agg_ref[0] + agg_ref[1]) / jnp.sqrt(
            jnp.float32(1.5))

    return pl.pallas_call(
        body,
        grid=(_N // _TB,),
        in_specs=[
            pl.BlockSpec((_TB, 16), lambda i: (i, 0)),
            pl.BlockSpec((_NC, _TB, 16), lambda i: (0, i, 0)),
        ],
        out_specs=pl.BlockSpec((_TB, 16), lambda i: (i, 0)),
        out_shape=jax.ShapeDtypeStruct((_N, 16), jnp.float32),
    )(xs4, agg4)


def kernel(x, edge_src, edge_dst, edge_attr,
           W_self_0, W_a_0, W_b_0,
           W_self_1, W_a_1, W_b_1,
           W_self_2, W_a_2, W_b_2,
           W_self_3, W_a_3, W_b_3,
           W_self_4, W_a_4, W_b_4):
    pad_e = _EPAD - _E
    pad_idx = jnp.arange(pad_e, dtype=jnp.int32) % _N
    src_p = jnp.concatenate([edge_src, pad_idx])
    dst_p = jnp.concatenate([edge_dst, pad_idx])
    attr_p = jnp.pad(edge_attr, ((0, pad_e), (0, 7)))
    wb = [jnp.pad(w, ((0, 7), (0, 0))) for w in (W_b_0, W_b_1, W_b_2, W_b_3)]
    wb4 = jnp.pad(W_b_4, ((0, 7), (0, 9)))
    wa4 = jnp.pad(W_a_4, ((0, 0), (0, 9)))
    ws4 = jnp.pad(W_self_4, ((0, 0), (0, 9)))
    x_p = jnp.pad(x, ((0, 0), (0, 7)))
    wa0 = jnp.pad(W_a_0, ((0, 7), (0, 0)))
    ws0 = jnp.pad(W_self_0, ((0, 7), (0, 0)))

    ebs = [_eb_one(attr_p, w) for w in (wb[0], wb[1], wb[2], wb[3], wb4)]

    xa, xs = _mm_pair(x_p, wa0, ws0)
    layer_ws = [(W_a_1, W_self_1), (W_a_2, W_self_2), (W_a_3, W_self_3),
                (wa4, ws4)]
    for i in range(4):
        agg = _edge_pass(xa, ebs[i], src_p, dst_p, 128)
        xa, xs = _layer_tc(xs, agg, layer_ws[i][0], layer_ws[i][1])
    agg4 = _edge_pass(xa, ebs[4], src_p, dst_p, 16)
    out16 = _final_combine(xs, agg4)
    return out16[:, :7]


# speedup vs baseline: 1.0896x; 1.0333x over previous
import jax
import jax.numpy as jnp
from jax import lax
from jax.experimental import pallas as pl
from jax.experimental.pallas import tpu as pltpu
from jax.experimental.pallas import tpu_sc as plsc

_N = 10000
_E = 320000
_NC = 2
_NS = 16
_NW = _NC * _NS
_CHUNK = 96
_CPW = 106
_EPAD = _NW * _CPW * _CHUNK
_RPT = 632
_NPAD = _NS * _RPT


def _edge_pass(xa, eb, src, dst, d):
    ngrp = d // 16
    packed = d == 128
    mesh = plsc.VectorSubcoreMesh(core_axis_name="c", subcore_axis_name="s")

    def body(xa_hbm, eb_hbm, src_hbm, dst_hbm, out_hbm,
             src_v0, dst_v0, rows_v0, eb_v0, gsem0, esem0,
             src_v1, dst_v1, rows_v1, eb_v1, gsem1, esem1,
             agg_sh):
        c = lax.axis_index("c")
        s = lax.axis_index("s")
        wid = s * _NC + c
        bufs = ((src_v0, dst_v0, rows_v0, eb_v0, gsem0, esem0),
                (src_v1, dst_v1, rows_v1, eb_v1, gsem1, esem1))

        z = jnp.zeros((16,), jnp.float32)

        def zbody(j, carry):
            for k in range(ngrp):
                rows_v0[j, pl.ds(k * 16, 16)] = z
            return carry

        lax.fori_loop(0, _CHUNK, zbody, 0)
        row0 = s * _RPT
        zfull, zrem = _RPT // _CHUNK, _RPT % _CHUNK
        for r in range(zfull):
            pltpu.sync_copy(rows_v0, agg_sh.at[pl.ds(row0 + r * _CHUNK, _CHUNK)])
        if zrem:
            pltpu.sync_copy(rows_v0.at[pl.ds(0, zrem)],
                            agg_sh.at[pl.ds(row0 + zfull * _CHUNK, zrem)])
        plsc.subcore_barrier()

        def prefetch(i, b):
            sv, dv, rv, ev, gs, es = bufs[b]
            base = (wid + i * _NW) * _CHUNK
            pltpu.sync_copy(src_hbm.at[pl.ds(base, _CHUNK)], sv)
            pltpu.sync_copy(dst_hbm.at[pl.ds(base, _CHUNK)], dv)
            pltpu.async_copy(xa_hbm.at[sv], rv, gs)
            pltpu.async_copy(eb_hbm.at[pl.ds(base, _CHUNK)], ev, es)

        def consume(b):
            sv, dv, rv, ev, gs, es = bufs[b]
            pltpu.make_async_copy(xa_hbm.at[sv], rv, gs).wait()
            pltpu.make_async_copy(eb_hbm.at[pl.ds(0, _CHUNK)], ev, es).wait()

            if packed:
                himask = jnp.uint32(0xFFFF0000)

                def mbody(j, mcarry):
                    for u in range(2):
                        r = 2 * j + u
                        for k in range(4):
                            w = ev[r, pl.ds(k * 16, 16)]
                            lo = plsc.bitcast(w << 16, jnp.float32)
                            hi = plsc.bitcast(w & himask, jnp.float32)
                            sl = pl.ds(k * 16, 16)
                            sh = pl.ds(64 + k * 16, 16)
                            rv[r, sl] = rv[r, sl] * lo
                            rv[r, sh] = rv[r, sh] * hi
                    return mcarry
            else:
                def mbody(j, mcarry):
                    for u in range(2):
                        for k in range(ngrp):
                            sl = pl.ds(k * 16, 16)
                            rv[2 * j + u, sl] = rv[2 * j + u, sl] * ev[2 * j + u, sl]
                    return mcarry

            lax.fori_loop(0, _CHUNK // 2, mbody, 0)
            pltpu.sync_copy(rv, agg_sh.at[dv], add=True)

        prefetch(jnp.int32(0), 0)

        def pair_body(p, carry):
            g = 2 * p
            prefetch(g + 1, 1)
            consume(0)
            prefetch(jnp.minimum(g + 2, _CPW - 1), 0)
            consume(1)
            return carry

        lax.fori_loop(0, _CPW // 2, pair_body, 0)
        sv, dv, rv, ev, gs, es = bufs[0]
        pltpu.make_async_copy(xa_hbm.at[sv], rv, gs).wait()
        pltpu.make_async_copy(eb_hbm.at[pl.ds(0, _CHUNK)], ev, es).wait()

        plsc.subcore_barrier()
        pltpu.sync_copy(agg_sh.at[pl.ds(row0, _RPT)],
                        out_hbm.at[c, pl.ds(row0, _RPT)])

    eb_buf = (pltpu.VMEM((_CHUNK, 64), jnp.uint32) if packed
              else pltpu.VMEM((_CHUNK, d), jnp.float32))
    buf_types = [
        pltpu.VMEM((_CHUNK,), jnp.int32),
        pltpu.VMEM((_CHUNK,), jnp.int32),
        pltpu.VMEM((_CHUNK, d), jnp.float32),
        eb_buf,
        pltpu.SemaphoreType.DMA,
        pltpu.SemaphoreType.DMA,
    ]
    call = pl.kernel(
        body,
        out_type=jax.ShapeDtypeStruct((_NC, _NPAD, d), jnp.float32),
        mesh=mesh,
        compiler_params=pltpu.CompilerParams(use_tc_tiling_on_sc=(d == 128),
                                             needs_layout_passes=not packed),
        scratch_types=buf_types + buf_types + [
            pltpu.VMEM_SHARED((_NPAD, d), jnp.float32),
        ],
    )
    return call(xa, eb, src, dst)


_EBLK = 2048


def _eb_one(attr_p, wb):
    dd = int(wb.shape[1])

    def body(attr_ref, w_ref, o_ref):
        o_ref[...] = jnp.dot(attr_ref[...], w_ref[...],
                             preferred_element_type=jnp.float32)

    return pl.pallas_call(
        body,
        grid=(_EPAD // _EBLK,),
        in_specs=[
            pl.BlockSpec((_EBLK, 16), lambda i: (i, 0)),
            pl.BlockSpec((16, dd), lambda i: (0, 0)),
        ],
        out_specs=pl.BlockSpec((_EBLK, dd), lambda i: (i, 0)),
        out_shape=jax.ShapeDtypeStruct((_EPAD, dd), jnp.float32),
    )(attr_p, wb)


def _eb_packed(attr_p, wb):
    def body(attr_ref, w_ref, o_ref):
        ebv = jnp.dot(attr_ref[...], w_ref[...],
                      preferred_element_type=jnp.float32)
        lo = lax.bitcast_convert_type(ebv[:, :64].astype(jnp.bfloat16),
                                      jnp.uint16).astype(jnp.uint32)
        hi = lax.bitcast_convert_type(ebv[:, 64:].astype(jnp.bfloat16),
                                      jnp.uint16).astype(jnp.uint32)
        o_ref[...] = lo | (hi << 16)

    return pl.pallas_call(
        body,
        grid=(_EPAD // _EBLK,),
        in_specs=[
            pl.BlockSpec((_EBLK, 16), lambda i: (i, 0)),
            pl.BlockSpec((16, 128), lambda i: (0, 0)),
        ],
        out_specs=pl.BlockSpec((_EBLK, 64), lambda i: (i, 0)),
        out_shape=jax.ShapeDtypeStruct((_EPAD, 64), jnp.uint32),
    )(attr_p, wb)


_TB = 2000


def _mm_pair(xin, wa, ws):
    din = int(xin.shape[1])
    da, dsf = int(wa.shape[1]), int(ws.shape[1])

    def body(x_ref, wa_ref, ws_ref, oa_ref, os_ref):
        x0 = x_ref[:, 0:1]
        oa_ref[...] = x0 * wa_ref[0:1, :]
        os_ref[...] = x0 * ws_ref[0:1, :]

    return pl.pallas_call(
        body,
        grid=(_N // _TB,),
        in_specs=[
            pl.BlockSpec((_TB, din), lambda i: (i, 0)),
            pl.BlockSpec((din, da), lambda i: (0, 0)),
            pl.BlockSpec((din, dsf), lambda i: (0, 0)),
        ],
        out_specs=[
            pl.BlockSpec((_TB, da), lambda i: (i, 0)),
            pl.BlockSpec((_TB, dsf), lambda i: (i, 0)),
        ],
        out_shape=[
            jax.ShapeDtypeStruct((_N, da), jnp.float32),
            jax.ShapeDtypeStruct((_N, dsf), jnp.float32),
        ],
    )(xin, wa, ws)


def _gate_block(h):
    sg = jax.nn.gelu(h[:, 0:32])
    st = jnp.tanh(h[:, 32:64])
    g = jax.nn.sigmoid(h[:, 64:80])
    row = lax.broadcasted_iota(jnp.int32, (16, 48), 0)
    col = lax.broadcasted_iota(jnp.int32, (16, 48), 1)
    r = (row == col // 3).astype(jnp.float32)
    gm = jnp.dot(g, r, preferred_element_type=jnp.float32, precision=lax.Precision.HIGHEST)
    return jnp.concatenate([sg, st, h[:, 80:128] * gm], axis=1)


def _layer_tc(xs_prev, agg, wa, ws):
    da, dsf = int(wa.shape[1]), int(ws.shape[1])

    def body(xs_ref, agg_ref, wa_ref, ws_ref, oa_ref, os_ref):
        h = (xs_ref[...] + agg_ref[0] + agg_ref[1]) / jnp.sqrt(jnp.float32(1.5))
        xv = _gate_block(h)
        oa_ref[...] = jnp.dot(xv, wa_ref[...], preferred_element_type=jnp.float32)
        os_ref[...] = jnp.dot(xv, ws_ref[...], preferred_element_type=jnp.float32)

    return pl.pallas_call(
        body,
        grid=(_N // _TB,),
        in_specs=[
            pl.BlockSpec((_TB, 128), lambda i: (i, 0)),
            pl.BlockSpec((_NC, _TB, 128), lambda i: (0, i, 0)),
            pl.BlockSpec((112, da), lambda i: (0, 0)),
            pl.BlockSpec((112, dsf), lambda i: (0, 0)),
        ],
        out_specs=[
            pl.BlockSpec((_TB, da), lambda i: (i, 0)),
            pl.BlockSpec((_TB, dsf), lambda i: (i, 0)),
        ],
        out_shape=[
            jax.ShapeDtypeStruct((_N, da), jnp.float32),
            jax.ShapeDtypeStruct((_N, dsf), jnp.float32),
        ],
    )(xs_prev, agg, wa, ws)


def _final_combine(xs4, agg4):
    def body(xs_ref, agg_ref, o_ref):
        o_ref[...] = (xs_ref[...] + agg_ref[0] + agg_ref[1]) / jnp.sqrt(
            jnp.float32(1.5))

    return pl.pallas_call(
        body,
        grid=(_N // _TB,),
        in_specs=[
            pl.BlockSpec((_TB, 16), lambda i: (i, 0)),
            pl.BlockSpec((_NC, _TB, 16), lambda i: (0, i, 0)),
        ],
        out_specs=pl.BlockSpec((_TB, 16), lambda i: (i, 0)),
        out_shape=jax.ShapeDtypeStruct((_N, 16), jnp.float32),
    )(xs4, agg4)


def kernel(x, edge_src, edge_dst, edge_attr,
           W_self_0, W_a_0, W_b_0,
           W_self_1, W_a_1, W_b_1,
           W_self_2, W_a_2, W_b_2,
           W_self_3, W_a_3, W_b_3,
           W_self_4, W_a_4, W_b_4):
    pad_e = _EPAD - _E
    pad_idx = jnp.arange(pad_e, dtype=jnp.int32) % _N
    src_p = jnp.concatenate([edge_src, pad_idx])
    dst_p = jnp.concatenate([edge_dst, pad_idx])
    attr_p = jnp.pad(edge_attr, ((0, pad_e), (0, 7)))
    wb = [jnp.pad(w, ((0, 7), (0, 0))) for w in (W_b_0, W_b_1, W_b_2, W_b_3)]
    wb4 = jnp.pad(W_b_4, ((0, 7), (0, 9)))
    wa4 = jnp.pad(W_a_4, ((0, 0), (0, 9)))
    ws4 = jnp.pad(W_self_4, ((0, 0), (0, 9)))
    x_p = jnp.pad(x, ((0, 0), (0, 7)))
    wa0 = jnp.pad(W_a_0, ((0, 7), (0, 0)))
    ws0 = jnp.pad(W_self_0, ((0, 7), (0, 0)))

    ebs = [_eb_packed(attr_p, w) for w in (wb[0], wb[1], wb[2], wb[3])]
    ebs.append(_eb_one(attr_p, wb4))

    xa, xs = _mm_pair(x_p, wa0, ws0)
    layer_ws = [(W_a_1, W_self_1), (W_a_2, W_self_2), (W_a_3, W_self_3),
                (wa4, ws4)]
    for i in range(4):
        agg = _edge_pass(xa, ebs[i], src_p, dst_p, 128)
        xa, xs = _layer_tc(xs, agg, layer_ws[i][0], layer_ws[i][1])
    agg4 = _edge_pass(xa, ebs[4], src_p, dst_p, 16)
    out16 = _final_combine(xs, agg4)
    return out16[:, :7]
